# baseline (speedup 1.0000x reference)
import functools

import jax
import jax.numpy as jnp
from jax import lax
from jax.experimental import pallas as pl
from jax.experimental.pallas import tpu as pltpu
from jax.experimental.pallas import tpu_sc as plsc

BATCH = 4096
SEQ = 200
EMBED = 128
NUM_CLASSES = 1000
VOCAB = 100000

NUM_CORES = 2
NUM_SUBCORES = 16
NUM_WORKERS = NUM_CORES * NUM_SUBCORES
SAMPLES_PER_WORKER = BATCH // NUM_WORKERS
CHUNK = 40
CHUNKS_PER_SAMPLE = SEQ // CHUNK
IDX_ROWS_PER_WORKER = SAMPLES_PER_WORKER * CHUNKS_PER_SAMPLE
LANES = 16
VECS = EMBED // LANES

V_SAMPLES = 90
A_SAMPLES = SAMPLES_PER_WORKER - V_SAMPLES
A_CHUNKS = A_SAMPLES * CHUNKS_PER_SAMPLE
GROUP = 30
NB = CHUNKS_PER_SAMPLE
NA = 2
ACC_ROWS = NUM_SUBCORES * A_SAMPLES


def _sc_pool(x2, table, dest):
    mesh = plsc.VectorSubcoreMesh(core_axis_name="c", subcore_axis_name="s")

    @functools.partial(
        pl.kernel,
        out_type=jax.ShapeDtypeStruct((BATCH, EMBED), jnp.float32),
        mesh=mesh,
        scratch_types=[
            pltpu.VMEM((IDX_ROWS_PER_WORKER, CHUNK), jnp.int32),
            pltpu.VMEM((A_SAMPLES, CHUNK), jnp.int32),
            pltpu.VMEM((NB, CHUNK, EMBED), jnp.float32),
            pltpu.VMEM((NA, CHUNK, EMBED), jnp.float32),
            pltpu.VMEM((GROUP, EMBED), jnp.float32),
            pltpu.VMEM((A_SAMPLES, EMBED), jnp.float32),
            pltpu.VMEM_SHARED((ACC_ROWS, EMBED), jnp.float32),
        ]
        + [pltpu.SemaphoreType.DMA] * (NB + 2 * NA),
        compiler_params=pltpu.CompilerParams(use_tc_tiling_on_sc=False),
    )
    def k(x_hbm, table_hbm, dest_hbm, out_hbm,
          idx_v, dest_v, rows_b, rows_a, acc_v, stage_v, acc_sh, *sems):
        sub = lax.axis_index("s")
        wid = sub * NUM_CORES + lax.axis_index("c")
        bsems = sems[:NB]
        gsems = sems[NB:NB + NA]
        asems = sems[NB + NA:]

        pltpu.sync_copy(
            x_hbm.at[pl.ds(wid * IDX_ROWS_PER_WORKER, IDX_ROWS_PER_WORKER)],
            idx_v,
        )
        pltpu.sync_copy(
            dest_hbm.at[pl.ds(sub * A_SAMPLES, A_SAMPLES)], dest_v
        )

        @pl.loop(0, A_SAMPLES)
        def _zero(s):
            for g in range(VECS):
                stage_v[s, pl.ds(LANES * g, LANES)] = jnp.zeros(
                    (LANES,), jnp.float32
                )

        pltpu.sync_copy(
            stage_v, acc_sh.at[pl.ds(sub * A_SAMPLES, A_SAMPLES)]
        )

        def issue_b(sample, c):
            pltpu.async_copy(
                table_hbm.at[idx_v.at[sample * CHUNKS_PER_SAMPLE + c]],
                rows_b.at[c],
                bsems[c],
            )

        def drain_b(c):
            pltpu.make_async_copy(
                table_hbm.at[pl.ds(0, CHUNK)], rows_b.at[c], bsems[c]
            ).wait()

        def accum_chunk(c, acc):
            @plsc.parallel_loop(0, CHUNK, unroll=8, carry=acc)
            def body(r, a):
                out = []
                for g in range(VECS):
                    w = rows_b[c, r, pl.ds(LANES * g, LANES)]
                    out.append(a[g] + w)
                return tuple(out)
            return body

        def issue_ga(t, j):
            sidx = lax.rem(t, A_SAMPLES)
            c = lax.div(t, A_SAMPLES)
            pltpu.async_copy(
                table_hbm.at[
                    idx_v.at[(V_SAMPLES + sidx) * CHUNKS_PER_SAMPLE + c]
                ],
                rows_a.at[j],
                gsems[j],
            )

        def wait_ga(j):
            pltpu.make_async_copy(
                table_hbm.at[pl.ds(0, CHUNK)], rows_a.at[j], gsems[j]
            ).wait()

        def issue_add(t, j):
            pltpu.async_copy(
                rows_a.at[j],
                acc_sh.at[dest_v.at[lax.rem(t, A_SAMPLES)]],
                asems[j],
                add=True,
            )

        def wait_add(t, j):
            pltpu.make_async_copy(
                rows_a.at[j],
                acc_sh.at[dest_v.at[lax.rem(t, A_SAMPLES)]],
                asems[j],
            ).wait()

        for c in range(NB):
            issue_b(0, c)
        for j in range(NA):
            issue_ga(j, j)

        def step(s, carry):
            t0 = 2 * s
            acc = tuple(jnp.zeros((LANES,), jnp.float32) for _ in range(VECS))
            for c in range(CHUNKS_PER_SAMPLE):
                drain_b(c)
                acc = accum_chunk(c, acc)

                @pl.when(s + 1 < V_SAMPLES)
                def _prefetch():
                    issue_b(s + 1, c)

                if c == 0:
                    wait_ga(0)
                    issue_add(t0, 0)
                elif c == 1:
                    wait_ga(1)
                    issue_add(t0 + 1, 1)
                elif c == 2:
                    wait_add(t0, 0)

                    @pl.when(t0 + 2 < A_CHUNKS)
                    def _g0():
                        issue_ga(t0 + 2, 0)
                elif c == 3:
                    wait_add(t0 + 1, 1)

                    @pl.when(t0 + 3 < A_CHUNKS)
                    def _g1():
                        issue_ga(t0 + 3, 1)

            s_mod = lax.rem(s, GROUP)
            for g in range(VECS):
                acc_v[s_mod, pl.ds(LANES * g, LANES)] = acc[g] * (1.0 / SEQ)

            @pl.when(s_mod == GROUP - 1)
            def _flush():
                grp = lax.div(s, GROUP)
                pltpu.sync_copy(
                    acc_v,
                    out_hbm.at[
                        pl.ds(wid * SAMPLES_PER_WORKER + grp * GROUP, GROUP)
                    ],
                )
            return carry

        lax.fori_loop(0, V_SAMPLES, step, 0)

        @pl.loop(2 * V_SAMPLES, A_CHUNKS, step=2)
        def _tail(t):
            for j in range(NA):
                wait_ga(j)
                issue_add(t + j, j)
                wait_add(t + j, j)

                @pl.when(t + j + NA < A_CHUNKS)
                def _g():
                    issue_ga(t + j + NA, j)

        pltpu.sync_copy(
            acc_sh.at[pl.ds(sub * A_SAMPLES, A_SAMPLES)], stage_v
        )

        @pl.loop(0, A_SAMPLES)
        def _scale(s):
            for g in range(VECS):
                sl = pl.ds(LANES * g, LANES)
                stage_v[s, sl] = stage_v[s, sl] * (1.0 / SEQ)

        pltpu.sync_copy(
            stage_v,
            out_hbm.at[
                pl.ds(wid * SAMPLES_PER_WORKER + V_SAMPLES, A_SAMPLES)
            ],
        )

    return k(x2, table, dest)


def _tc_matmul(pooled, W, b2):
    BB = 512

    def body(p_ref, w_ref, b_ref, o_ref):
        o_ref[...] = (
            jnp.dot(p_ref[...], w_ref[...], preferred_element_type=jnp.float32)
            + b_ref[...]
        )

    return pl.pallas_call(
        body,
        grid=(BATCH // BB,),
        in_specs=[
            pl.BlockSpec((BB, EMBED), lambda i: (i, 0)),
            pl.BlockSpec((EMBED, NUM_CLASSES), lambda i: (0, 0)),
            pl.BlockSpec((1, NUM_CLASSES), lambda i: (0, 0)),
        ],
        out_specs=pl.BlockSpec((BB, NUM_CLASSES), lambda i: (i, 0)),
        out_shape=jax.ShapeDtypeStruct((BATCH, NUM_CLASSES), jnp.float32),
    )(pooled, W, b2)


def kernel(x, table, W, b):
    x2 = x.astype(jnp.int32).reshape(BATCH * CHUNKS_PER_SAMPLE, CHUNK)
    dest = jnp.broadcast_to(
        jnp.arange(ACC_ROWS, dtype=jnp.int32)[:, None], (ACC_ROWS, CHUNK)
    )
    pooled = _sc_pool(x2, table, dest)
    return _tc_matmul(pooled, W, b.reshape(1, NUM_CLASSES))
